# baseline (speedup 1.0000x reference)
import functools

import jax
import jax.numpy as jnp
from jax import lax
from jax.experimental import pallas as pl
from jax.experimental.pallas import tpu as pltpu

L = 3
D = 128
N_USER = 5000
N_ITEM = 5000
E = 320000
TMAX = 50
B = 128


def _seg_softmax(e, seg, num):
    m = jax.ops.segment_max(e, seg, num_segments=num)
    m = jnp.where(jnp.isfinite(m), m, 0.0)
    ex = jnp.exp(e - m[seg])
    s = jax.ops.segment_sum(ex, seg, num_segments=num)
    return ex / (s[seg] + 1e-9)


def _score_kernel(u_ref, it_ref, o_ref):
    o_ref[...] = lax.dot_general(
        u_ref[...], it_ref[...], (((1,), (1,)), ((), ())),
        preferred_element_type=jnp.float32)


def _score_matmul(unified_b, item_emb):
    NBLK = 2048
    n = item_emb.shape[0]
    grid = (pl.cdiv(n, NBLK),)
    return pl.pallas_call(
        _score_kernel,
        grid=grid,
        in_specs=[
            pl.BlockSpec((B, D), lambda j: (0, 0)),
            pl.BlockSpec((NBLK, D), lambda j: (j, 0)),
        ],
        out_specs=pl.BlockSpec((B, NBLK), lambda j: (0, j)),
        out_shape=jax.ShapeDtypeStruct((B, n), jnp.float32),
    )(unified_b, item_emb)


def kernel(user_ids, item_ids, edge_src_item, edge_dst_user, edge_order_u,
           edge_order_i, last_item_per_user, last_user_per_item, batch_users,
           user_emb, item_emb, user_weight, item_weight, agg_gate_u,
           agg_gate_i, u_te, u_te_k, i_te, i_te_k, last_w_u, last_w_i,
           upd_u, upd_i, unified_map):
    scale = jnp.sqrt(jnp.asarray(D, jnp.float32))
    src = edge_src_item
    dst = edge_dst_user
    t_u = edge_order_u
    t_i = edge_order_i
    last_it = last_item_per_user
    last_us = last_user_per_item

    u0 = user_emb[user_ids]
    i0 = item_emb[item_ids]
    u, it = u0, i0
    layer_outs = []
    for l in range(L):
        uh = u @ user_weight[l]
        ih = it @ item_weight[l]
        ih_src = ih[src]
        e_u = jnp.sum((ih_src + u_te_k[l][t_u]) * uh[dst], axis=-1) / scale
        a_u = _seg_softmax(e_u, dst, N_USER)
        h_long_u = jax.ops.segment_sum(
            a_u[:, None] * (ih_src + u_te[l][t_u]), dst, num_segments=N_USER)
        q_last = (ih[last_it] @ last_w_u[l])[dst]
        e2 = jnp.sum(q_last * ih_src, axis=-1) / scale
        a2 = _seg_softmax(e2, dst, N_USER)
        h_short_u = jax.ops.segment_sum(a2[:, None] * ih_src, dst,
                                        num_segments=N_USER)
        h_u_now = jnp.concatenate([h_long_u, h_short_u], axis=-1) @ agg_gate_u[l]
        uh_dst = uh[dst]
        e_i = jnp.sum((uh_dst + i_te_k[l][t_i]) * ih[src], axis=-1) / scale
        a_i = _seg_softmax(e_i, src, N_ITEM)
        h_long_i = jax.ops.segment_sum(
            a_i[:, None] * (uh_dst + i_te[l][t_i]), src, num_segments=N_ITEM)
        q_last_i = (uh[last_us] @ last_w_i[l])[src]
        e2i = jnp.sum(q_last_i * uh_dst, axis=-1) / scale
        a2i = _seg_softmax(e2i, src, N_ITEM)
        h_short_i = jax.ops.segment_sum(a2i[:, None] * uh_dst, src,
                                        num_segments=N_ITEM)
        h_i_now = jnp.concatenate([h_long_i, h_short_i], axis=-1) @ agg_gate_i[l]
        u = jnp.tanh(jnp.concatenate([h_u_now, u], axis=-1) @ upd_u[l])
        it = jnp.tanh(jnp.concatenate([h_i_now, it], axis=-1) @ upd_i[l])
        layer_outs.append(u)
    layer_outs.append(i0[last_it])
    unified = jnp.concatenate(layer_outs, axis=-1) @ unified_map
    unified_b = unified[batch_users]
    return _score_matmul(unified_b, item_emb)


# speedup vs baseline: 3.8499x; 3.8499x over previous
import functools

import jax
import jax.numpy as jnp
from jax import lax
from jax.experimental import pallas as pl
from jax.experimental.pallas import tpu as pltpu
from jax.experimental.pallas import tpu_sc as plsc

L = 3
D = 128
N = 5000
NP = 5120
E = 320000
TMAX = 50
B = 128

NC = 2
NS = 16
NW = NC * NS
CH = 80
EPW = E // NW
NCHUNK = EPW // CH
DNR = NP // 16
RB = 1024
INV_SCALE = 1.0 / (D ** 0.5)

_mesh = plsc.VectorSubcoreMesh(core_axis_name="c", subcore_axis_name="s")

_GDN = lax.GatherDimensionNumbers(offset_dims=(), collapsed_slice_dims=(0,),
                                  start_index_map=(0,))


def _lanesum(v):
    ii = lax.iota(jnp.int32, 16)
    for k in (1, 2, 4, 8):
        p = lax.gather(v, jnp.bitwise_xor(ii, k)[:, None], _GDN,
                       slice_sizes=(1,),
                       mode=lax.GatherScatterMode.PROMISE_IN_BOUNDS)
        v = v + p
    return v


_lanesum_i = _lanesum


def _f32(x):
    return x.astype(jnp.float32)


@functools.lru_cache(maxsize=None)
def _rows_fn(V, BN):
    bpw = BN // NW
    cw = min(bpw, CH)
    nk = bpw // cw

    @functools.partial(
        pl.kernel,
        out_type=jax.ShapeDtypeStruct((BN, D), jnp.float32),
        mesh=_mesh,
        scratch_types=[
            pltpu.VMEM((cw,), jnp.int32),
            pltpu.VMEM((cw, D), jnp.float32),
            pltpu.SemaphoreType.DMA,
        ],
    )
    def rows_kernel(tab_h, idx_h, out_h, idxv, rbuf, sem):
        wid = lax.axis_index("s") * NC + lax.axis_index("c")
        base = wid * bpw
        for k in range(nk):
            off = base + k * cw
            pltpu.sync_copy(idx_h.at[pl.ds(off, cw)], idxv)
            pltpu.async_copy(tab_h.at[idxv], rbuf, sem).wait()
            pltpu.sync_copy(rbuf, out_h.at[pl.ds(off, cw)])

    return rows_kernel


def _rows(table, idx):
    return _rows_fn(table.shape[0], idx.shape[0])(table, idx)


DR = 48


@functools.lru_cache(maxsize=None)
def _agg_fn():
    scratch = [
        pltpu.VMEM((CH,), jnp.int32),
        pltpu.VMEM((CH,), jnp.int32),
        pltpu.VMEM((CH,), jnp.int32),
        pltpu.VMEM((CH,), jnp.int32),
        pltpu.VMEM((CH, D), jnp.float32),
        pltpu.VMEM((CH, D), jnp.float32),
        pltpu.VMEM((CH, D), jnp.float32),
        pltpu.VMEM((CH, D), jnp.float32),
        pltpu.VMEM((CH, D), jnp.float32),
        pltpu.VMEM((CH, D), jnp.float32),
        pltpu.VMEM_SHARED((NP, D), jnp.float32),
        pltpu.VMEM_SHARED((DR, D), jnp.float32),
        pltpu.SemaphoreType.DMA,
        pltpu.SemaphoreType.DMA,
        pltpu.SemaphoreType.DMA,
        pltpu.SemaphoreType.DMA,
        pltpu.SemaphoreType.DMA,
    ]
    out_type = (jax.ShapeDtypeStruct((NC, NP, D), jnp.float32),
                jax.ShapeDtypeStruct((NC, DR, D), jnp.float32))

    def body(a_h, q_h, tk_h, tv_h, ia_h, iq_h, tt_h, num_h, den_h,
             iav, iqv, ttv, iqv2, arows, qrows, tkrows,
             tvrows, vbuf, dbuf, num_sh, den_sh,
             semA, semQ, semK, semV, semI):
        cid = lax.axis_index("c")
        sid = lax.axis_index("s")
        wid = sid * NC + cid
        ebase = wid * EPW
        iota = lax.iota(jnp.int32, 16)
        zero16 = jnp.zeros((16,), jnp.float32)

        def zrow(r, _):
            for s in range(D // 16):
                vbuf[r, pl.ds(s * 16, 16)] = zero16
            return 0
        lax.fori_loop(0, CH, zrow, 0)

        zb = sid * (NP // NS)
        for k in range(NP // NS // CH):
            pltpu.sync_copy(vbuf, num_sh.at[pl.ds(zb + k * CH, CH)])
        @pl.when(sid < DR // 8)
        def _():
            pltpu.sync_copy(vbuf.at[pl.ds(0, 8)],
                            den_sh.at[pl.ds(sid * 8, 8)])
        plsc.subcore_barrier()

        def chunk(c, _):
            eb = ebase + c * CH
            gi = pltpu.async_copy(ia_h.at[pl.ds(eb, CH)], iav, semI)
            gq0 = pltpu.async_copy(iq_h.at[pl.ds(eb, CH)], iqv, semQ)
            gt = pltpu.async_copy(tt_h.at[pl.ds(eb, CH)], ttv, semK)
            gi.wait()
            gq0.wait()
            gt.wait()
            for i in range(CH // 16):
                iq16i = iqv[pl.ds(i * 16, 16)]
                iqv2[pl.ds(i * 16, 16)] = lax.shift_right_logical(iq16i, 7)
            ga = pltpu.async_copy(a_h.at[iav], arows, semA)
            gq = pltpu.async_copy(q_h.at[iqv], qrows, semQ)
            gk = pltpu.async_copy(tk_h.at[ttv], tkrows, semK)
            gv = pltpu.async_copy(tv_h.at[ttv], tvrows, semV)
            ga.wait()
            gq.wait()
            gk.wait()
            gv.wait()

            for g in range(CH // 16):
                iq16 = iqv[pl.ds(g * 16, 16)]

                def edge(j, _):
                    jj = g * 16 + j
                    acc1 = zero16
                    for s in range(8):
                        sl = pl.ds(s * 16, 16)
                        acc1 = acc1 + ((arows[jj, sl] + tkrows[jj, sl])
                                       * qrows[jj, sl])
                    w1 = jnp.exp(_lanesum(acc1) * INV_SCALE)
                    for s in range(8):
                        sl = pl.ds(s * 16, 16)
                        vbuf[jj, sl] = (arows[jj, sl]
                                        + tvrows[jj, sl]) * w1
                    sel = jnp.where(iota == j, iq16, 0)
                    s16 = _lanesum_i(sel)
                    c2 = jnp.bitwise_and(s16, D - 1)
                    for s in range(8):
                        m = iota + s * 16 == c2
                        dbuf[jj, pl.ds(s * 16, 16)] = jnp.where(m, w1, 0.0)
                    return 0
                lax.fori_loop(0, 16, edge, 0)

            pltpu.sync_copy(vbuf, num_sh.at[iqv], add=True)
            pltpu.sync_copy(dbuf, den_sh.at[iqv2], add=True)
            return 0
        lax.fori_loop(0, NCHUNK, chunk, 0)

        plsc.subcore_barrier()

        @pl.when(sid == 0)
        def _():
            pltpu.sync_copy(num_sh, num_h.at[cid])
            pltpu.sync_copy(den_sh, den_h.at[cid])

    return pl.kernel(body, out_type=out_type, mesh=_mesh,
                     scratch_types=scratch)


def _agg2(A, Q, TEk, TEv, ia, iq, tt):
    return _agg_fn()(A, Q, TEk, TEv, ia, iq, tt)


def _lin_body(u_ref, it_ref, g1_ref, g2_ref, wu_ref, wi_ref, lwu_ref,
              lwi_ref, uh_ref, ih_ref, qlu_ref, qli_ref):
    f32 = jnp.float32
    uh_ref[...] = jnp.dot(u_ref[...], wu_ref[...], preferred_element_type=f32)
    ih_ref[...] = jnp.dot(it_ref[...], wi_ref[...],
                          preferred_element_type=f32)
    mu = jnp.dot(wi_ref[...], lwu_ref[...], preferred_element_type=f32)
    mi = jnp.dot(wu_ref[...], lwi_ref[...], preferred_element_type=f32)
    qlu_ref[...] = jnp.dot(g1_ref[...], mu, preferred_element_type=f32)
    qli_ref[...] = jnp.dot(g2_ref[...], mi, preferred_element_type=f32)


def _lin(u, it, g1, g2, wu, wi, lwu, lwi):
    row = pl.BlockSpec((RB, D), lambda r: (r, 0))
    wsp = pl.BlockSpec((D, D), lambda r: (0, 0))
    sds = jax.ShapeDtypeStruct((NP, D), jnp.float32)
    return pl.pallas_call(
        _lin_body,
        grid=(NP // RB,),
        in_specs=[row, row, row, row, wsp, wsp, wsp, wsp],
        out_specs=[row, row, row, row],
        out_shape=[sds, sds, sds, sds],
    )(u, it, g1, g2, wu, wi, lwu, lwi)


def _upd_body(nLu, dLu, nSu, dSu, nLi, dLi, nSi, dSi, agu_ref, agi_ref,
              upu_ref, upi_ref, u_ref, it_ref, un_ref, itn_ref):
    f32 = jnp.float32
    ii8 = lax.broadcasted_iota(jnp.int32, (RB, RB // D), 0)
    aa8 = lax.broadcasted_iota(jnp.int32, (RB, RB // D), 1)
    ohr = (ii8 // D == aa8).astype(f32)
    iiD = lax.broadcasted_iota(jnp.int32, (RB, D), 0)
    bbD = lax.broadcasted_iota(jnp.int32, (RB, D), 1)
    ohc = (iiD % D == bbD).astype(f32)

    def h(nref, dref):
        n = nref[...]
        d = dref[...]
        rep = jnp.dot(ohr, d[0] + d[1], preferred_element_type=f32)
        dd = jnp.sum(rep * ohc, axis=1, keepdims=True)
        return (n[0] + n[1]) / (dd + 1e-9)

    hLu = h(nLu, dLu)
    hSu = h(nSu, dSu)
    hLi = h(nLi, dLi)
    hSi = h(nSi, dSi)
    agu = agu_ref[...]
    agi = agi_ref[...]
    upu = upu_ref[...]
    upi = upi_ref[...]
    hu = (jnp.dot(hLu, agu[:D], preferred_element_type=f32)
          + jnp.dot(hSu, agu[D:], preferred_element_type=f32))
    hi = (jnp.dot(hLi, agi[:D], preferred_element_type=f32)
          + jnp.dot(hSi, agi[D:], preferred_element_type=f32))
    un_ref[...] = jnp.tanh(
        jnp.dot(hu, upu[:D], preferred_element_type=f32)
        + jnp.dot(u_ref[...], upu[D:], preferred_element_type=f32))
    itn_ref[...] = jnp.tanh(
        jnp.dot(hi, upi[:D], preferred_element_type=f32)
        + jnp.dot(it_ref[...], upi[D:], preferred_element_type=f32))


def _upd(aLu, aSu, aLi, aSi, agu, agi, upu, upi, u, it):
    nsp = pl.BlockSpec((NC, RB, D), lambda r: (0, r, 0))
    dsp = pl.BlockSpec((NC, RB // D, D), lambda r: (0, r, 0))
    wsp = pl.BlockSpec((2 * D, D), lambda r: (0, 0))
    row = pl.BlockSpec((RB, D), lambda r: (r, 0))
    sds = jax.ShapeDtypeStruct((NP, D), jnp.float32)
    return pl.pallas_call(
        _upd_body,
        grid=(NP // RB,),
        in_specs=[nsp, dsp, nsp, dsp, nsp, dsp, nsp, dsp,
                  wsp, wsp, wsp, wsp, row, row],
        out_specs=[row, row],
        out_shape=[sds, sds],
    )(aLu[0], aLu[1], aSu[0], aSu[1], aLi[0], aLi[1], aSi[0], aSi[1],
      agu, agi, upu, upi, u, it)


def _score_body(b1_ref, b2_ref, b3_ref, bl_ref, w_ref, item_ref, out_ref):
    f32 = jnp.float32
    w = w_ref[...]
    ub = (jnp.dot(b1_ref[...][:B], w[:D], preferred_element_type=f32)
          + jnp.dot(b2_ref[...][:B], w[D:2 * D], preferred_element_type=f32)
          + jnp.dot(b3_ref[...][:B], w[2 * D:3 * D],
                    preferred_element_type=f32)
          + jnp.dot(bl_ref[...][:B], w[3 * D:], preferred_element_type=f32))
    out_ref[...] = lax.dot_general(ub, item_ref[...],
                                   (((1,), (1,)), ((), ())),
                                   preferred_element_type=f32)


def _score(b1, b2, b3, bl, wuni, item_emb):
    n_item = item_emb.shape[0]
    IB = 2048
    bsp = pl.BlockSpec((2 * B, D), lambda j: (0, 0))
    return pl.pallas_call(
        _score_body,
        grid=(pl.cdiv(n_item, IB),),
        in_specs=[bsp, bsp, bsp, bsp,
                  pl.BlockSpec(((L + 1) * D, D), lambda j: (0, 0)),
                  pl.BlockSpec((IB, D), lambda j: (j, 0))],
        out_specs=pl.BlockSpec((B, IB), lambda j: (0, j)),
        out_shape=jax.ShapeDtypeStruct((B, n_item), jnp.float32),
    )(b1, b2, b3, bl, wuni, item_emb)


def _pad_idx(idx, n):
    pad = jnp.zeros((n - idx.shape[0],), jnp.int32)
    return jnp.concatenate([idx.astype(jnp.int32), pad])


def kernel(user_ids, item_ids, edge_src_item, edge_dst_user, edge_order_u,
           edge_order_i, last_item_per_user, last_user_per_item, batch_users,
           user_emb, item_emb, user_weight, item_weight, agg_gate_u,
           agg_gate_i, u_te, u_te_k, i_te, i_te_k, last_w_u, last_w_i,
           upd_u, upd_i, unified_map):
    src = edge_src_item.astype(jnp.int32)
    dst = edge_dst_user.astype(jnp.int32)
    t_u = edge_order_u.astype(jnp.int32)
    t_i = edge_order_i.astype(jnp.int32)
    last_it = _pad_idx(last_item_per_user, NP)
    last_us = _pad_idx(last_user_per_item, NP)

    u = _rows(_f32(user_emb), _pad_idx(user_ids, NP))
    i0 = _rows(_f32(item_emb), _pad_idx(item_ids, NP))
    it = i0

    layer_u = []
    for l in range(L):
        g1 = _rows(it, last_it)
        g2 = _rows(u, last_us)
        uh, ih, qlu, qli = _lin(u, it, g1, g2, user_weight[l],
                                item_weight[l], last_w_u[l], last_w_i[l])
        zte = jnp.zeros((TMAX, D), jnp.float32)
        aLu = _agg2(ih, uh, u_te_k[l], u_te[l], src, dst, t_u)
        aSu = _agg2(ih, qlu, zte, zte, src, dst, t_u)
        aLi = _agg2(uh, ih, i_te_k[l], i_te[l], dst, src, t_i)
        aSi = _agg2(uh, qli, zte, zte, dst, src, t_i)
        u, it = _upd(aLu, aSu, aLi, aSi,
                     agg_gate_u[l], agg_gate_i[l], upd_u[l], upd_i[l],
                     u, it)
        layer_u.append(u)

    bidx = jnp.concatenate([batch_users, batch_users]).astype(jnp.int32)
    lidx = jnp.concatenate([last_item_per_user[batch_users]] * 2)
    b1 = _rows(layer_u[0], bidx)
    b2 = _rows(layer_u[1], bidx)
    b3 = _rows(layer_u[2], bidx)
    bl = _rows(i0, lidx.astype(jnp.int32))
    return _score(b1, b2, b3, bl, unified_map, _f32(item_emb))


# speedup vs baseline: 4.3722x; 1.1356x over previous
import functools

import jax
import jax.numpy as jnp
from jax import lax
from jax.experimental import pallas as pl
from jax.experimental.pallas import tpu as pltpu
from jax.experimental.pallas import tpu_sc as plsc

L = 3
D = 128
N = 5000
NP = 5120
E = 320000
TMAX = 50
B = 128

NC = 2
NS = 16
NW = NC * NS
CH = 80
EPW = E // NW
NCHUNK = EPW // CH
DNR = NP // 16
RB = 1024
INV_SCALE = 1.0 / (D ** 0.5)

_mesh = plsc.VectorSubcoreMesh(core_axis_name="c", subcore_axis_name="s")

_GDN = lax.GatherDimensionNumbers(offset_dims=(), collapsed_slice_dims=(0,),
                                  start_index_map=(0,))


def _lanesum(v):
    ii = lax.iota(jnp.int32, 16)
    for k in (1, 2, 4, 8):
        p = lax.gather(v, jnp.bitwise_xor(ii, k)[:, None], _GDN,
                       slice_sizes=(1,),
                       mode=lax.GatherScatterMode.PROMISE_IN_BOUNDS)
        v = v + p
    return v


_lanesum_i = _lanesum


def _f32(x):
    return x.astype(jnp.float32)


@functools.lru_cache(maxsize=None)
def _rows_fn(V, BN):
    bpw = BN // NW
    cw = min(bpw, CH)
    nk = bpw // cw

    @functools.partial(
        pl.kernel,
        out_type=jax.ShapeDtypeStruct((BN, D), jnp.float32),
        mesh=_mesh,
        scratch_types=[
            pltpu.VMEM((cw,), jnp.int32),
            pltpu.VMEM((cw, D), jnp.float32),
            pltpu.SemaphoreType.DMA,
        ],
    )
    def rows_kernel(tab_h, idx_h, out_h, idxv, rbuf, sem):
        wid = lax.axis_index("s") * NC + lax.axis_index("c")
        base = wid * bpw
        for k in range(nk):
            off = base + k * cw
            pltpu.sync_copy(idx_h.at[pl.ds(off, cw)], idxv)
            pltpu.async_copy(tab_h.at[idxv], rbuf, sem).wait()
            pltpu.sync_copy(rbuf, out_h.at[pl.ds(off, cw)])

    return rows_kernel


def _rows(table, idx):
    return _rows_fn(table.shape[0], idx.shape[0])(table, idx)


DR = 48


@functools.lru_cache(maxsize=None)
def _agg_fn(use_te=True):
    scratch = [
        pltpu.VMEM((CH,), jnp.int32),
        pltpu.VMEM((CH,), jnp.int32),
        pltpu.VMEM((CH,), jnp.int32),
        pltpu.VMEM((CH,), jnp.int32),
        pltpu.VMEM((CH, D), jnp.float32),
        pltpu.VMEM((CH, D), jnp.float32),
        pltpu.VMEM((CH, D), jnp.float32),
        pltpu.VMEM((CH, D), jnp.float32),
        pltpu.VMEM((CH, D), jnp.float32),
        pltpu.VMEM((CH, D), jnp.float32),
        pltpu.VMEM_SHARED((NP, D), jnp.float32),
        pltpu.VMEM_SHARED((DR, D), jnp.float32),
        pltpu.SemaphoreType.DMA,
        pltpu.SemaphoreType.DMA,
        pltpu.SemaphoreType.DMA,
        pltpu.SemaphoreType.DMA,
        pltpu.SemaphoreType.DMA,
    ]
    if not use_te:
        del scratch[6:8]
        del scratch[2]
        del scratch[-2:]
    out_type = (jax.ShapeDtypeStruct((NC, NP, D), jnp.float32),
                jax.ShapeDtypeStruct((NC, DR, D), jnp.float32))

    def body(*refs):
        if use_te:
            (a_h, q_h, tk_h, tv_h, ia_h, iq_h, tt_h, num_h, den_h,
             iav, iqv, ttv, iqv2, arows, qrows, tkrows,
             tvrows, vbuf, dbuf, num_sh, den_sh,
             semA, semQ, semK, semV, semI) = refs
        else:
            (a_h, q_h, ia_h, iq_h, num_h, den_h,
             iav, iqv, iqv2, arows, qrows, vbuf, dbuf, num_sh, den_sh,
             semA, semQ, semI) = refs
        cid = lax.axis_index("c")
        sid = lax.axis_index("s")
        wid = sid * NC + cid
        ebase = wid * EPW
        iota = lax.iota(jnp.int32, 16)
        zero16 = jnp.zeros((16,), jnp.float32)

        def zrow(r, _):
            for s in range(D // 16):
                vbuf[r, pl.ds(s * 16, 16)] = zero16
            return 0
        lax.fori_loop(0, CH, zrow, 0)

        zb = sid * (NP // NS)
        for k in range(NP // NS // CH):
            pltpu.sync_copy(vbuf, num_sh.at[pl.ds(zb + k * CH, CH)])
        @pl.when(sid < DR // 8)
        def _():
            pltpu.sync_copy(vbuf.at[pl.ds(0, 8)],
                            den_sh.at[pl.ds(sid * 8, 8)])
        plsc.subcore_barrier()

        def chunk(c, _):
            eb = ebase + c * CH
            gi = pltpu.async_copy(ia_h.at[pl.ds(eb, CH)], iav, semI)
            gq0 = pltpu.async_copy(iq_h.at[pl.ds(eb, CH)], iqv, semQ)
            if use_te:
                gt = pltpu.async_copy(tt_h.at[pl.ds(eb, CH)], ttv, semK)
            gi.wait()
            gq0.wait()
            if use_te:
                gt.wait()
            for i in range(CH // 16):
                iq16i = iqv[pl.ds(i * 16, 16)]
                iqv2[pl.ds(i * 16, 16)] = lax.shift_right_logical(iq16i, 7)
            ga = pltpu.async_copy(a_h.at[iav], arows, semA)
            gq = pltpu.async_copy(q_h.at[iqv], qrows, semQ)
            if use_te:
                gk = pltpu.async_copy(tk_h.at[ttv], tkrows, semK)
                gv = pltpu.async_copy(tv_h.at[ttv], tvrows, semV)
            ga.wait()
            gq.wait()
            if use_te:
                gk.wait()
                gv.wait()

            for g in range(CH // 16):
                iq16 = iqv[pl.ds(g * 16, 16)]

                def edge(j, _):
                    jj = g * 16 + j
                    acc1 = zero16
                    for s in range(8):
                        sl = pl.ds(s * 16, 16)
                        kv = arows[jj, sl]
                        if use_te:
                            kv = kv + tkrows[jj, sl]
                        acc1 = acc1 + kv * qrows[jj, sl]
                    w1 = jnp.exp(_lanesum(acc1) * INV_SCALE)
                    for s in range(8):
                        sl = pl.ds(s * 16, 16)
                        vv = arows[jj, sl]
                        if use_te:
                            vv = vv + tvrows[jj, sl]
                        vbuf[jj, sl] = vv * w1
                    sel = jnp.where(iota == j, iq16, 0)
                    s16 = _lanesum_i(sel)
                    c2 = jnp.bitwise_and(s16, D - 1)
                    for s in range(8):
                        m = iota + s * 16 == c2
                        dbuf[jj, pl.ds(s * 16, 16)] = jnp.where(m, w1, 0.0)
                    return 0
                lax.fori_loop(0, 16, edge, 0)

            pltpu.sync_copy(vbuf, num_sh.at[iqv], add=True)
            pltpu.sync_copy(dbuf, den_sh.at[iqv2], add=True)
            return 0
        lax.fori_loop(0, NCHUNK, chunk, 0)

        plsc.subcore_barrier()

        @pl.when(sid == 0)
        def _():
            pltpu.sync_copy(num_sh, num_h.at[cid])
            pltpu.sync_copy(den_sh, den_h.at[cid])

    return pl.kernel(body, out_type=out_type, mesh=_mesh,
                     scratch_types=scratch)


def _agg2(A, Q, TEk, TEv, ia, iq, tt):
    return _agg_fn(True)(A, Q, TEk, TEv, ia, iq, tt)


def _agg2s(A, Q, ia, iq):
    return _agg_fn(False)(A, Q, ia, iq)


def _lin_body(u_ref, it_ref, g1_ref, g2_ref, wu_ref, wi_ref, lwu_ref,
              lwi_ref, uh_ref, ih_ref, qlu_ref, qli_ref):
    f32 = jnp.float32
    uh_ref[...] = jnp.dot(u_ref[...], wu_ref[...], preferred_element_type=f32)
    ih_ref[...] = jnp.dot(it_ref[...], wi_ref[...],
                          preferred_element_type=f32)
    mu = jnp.dot(wi_ref[...], lwu_ref[...], preferred_element_type=f32)
    mi = jnp.dot(wu_ref[...], lwi_ref[...], preferred_element_type=f32)
    qlu_ref[...] = jnp.dot(g1_ref[...], mu, preferred_element_type=f32)
    qli_ref[...] = jnp.dot(g2_ref[...], mi, preferred_element_type=f32)


def _lin(u, it, g1, g2, wu, wi, lwu, lwi):
    row = pl.BlockSpec((RB, D), lambda r: (r, 0))
    wsp = pl.BlockSpec((D, D), lambda r: (0, 0))
    sds = jax.ShapeDtypeStruct((NP, D), jnp.float32)
    return pl.pallas_call(
        _lin_body,
        grid=(NP // RB,),
        in_specs=[row, row, row, row, wsp, wsp, wsp, wsp],
        out_specs=[row, row, row, row],
        out_shape=[sds, sds, sds, sds],
    )(u, it, g1, g2, wu, wi, lwu, lwi)


def _upd_body(nLu, dLu, nSu, dSu, nLi, dLi, nSi, dSi, agu_ref, agi_ref,
              upu_ref, upi_ref, u_ref, it_ref, un_ref, itn_ref):
    f32 = jnp.float32
    ii8 = lax.broadcasted_iota(jnp.int32, (RB, RB // D), 0)
    aa8 = lax.broadcasted_iota(jnp.int32, (RB, RB // D), 1)
    ohr = (ii8 // D == aa8).astype(f32)
    iiD = lax.broadcasted_iota(jnp.int32, (RB, D), 0)
    bbD = lax.broadcasted_iota(jnp.int32, (RB, D), 1)
    ohc = (iiD % D == bbD).astype(f32)

    def h(nref, dref):
        n = nref[...]
        d = dref[...]
        rep = jnp.dot(ohr, d[0] + d[1], preferred_element_type=f32)
        dd = jnp.sum(rep * ohc, axis=1, keepdims=True)
        return (n[0] + n[1]) / (dd + 1e-9)

    hLu = h(nLu, dLu)
    hSu = h(nSu, dSu)
    hLi = h(nLi, dLi)
    hSi = h(nSi, dSi)
    agu = agu_ref[...]
    agi = agi_ref[...]
    upu = upu_ref[...]
    upi = upi_ref[...]
    hu = (jnp.dot(hLu, agu[:D], preferred_element_type=f32)
          + jnp.dot(hSu, agu[D:], preferred_element_type=f32))
    hi = (jnp.dot(hLi, agi[:D], preferred_element_type=f32)
          + jnp.dot(hSi, agi[D:], preferred_element_type=f32))
    un_ref[...] = jnp.tanh(
        jnp.dot(hu, upu[:D], preferred_element_type=f32)
        + jnp.dot(u_ref[...], upu[D:], preferred_element_type=f32))
    itn_ref[...] = jnp.tanh(
        jnp.dot(hi, upi[:D], preferred_element_type=f32)
        + jnp.dot(it_ref[...], upi[D:], preferred_element_type=f32))


def _upd(aLu, aSu, aLi, aSi, agu, agi, upu, upi, u, it):
    nsp = pl.BlockSpec((NC, RB, D), lambda r: (0, r, 0))
    dsp = pl.BlockSpec((NC, RB // D, D), lambda r: (0, r, 0))
    wsp = pl.BlockSpec((2 * D, D), lambda r: (0, 0))
    row = pl.BlockSpec((RB, D), lambda r: (r, 0))
    sds = jax.ShapeDtypeStruct((NP, D), jnp.float32)
    return pl.pallas_call(
        _upd_body,
        grid=(NP // RB,),
        in_specs=[nsp, dsp, nsp, dsp, nsp, dsp, nsp, dsp,
                  wsp, wsp, wsp, wsp, row, row],
        out_specs=[row, row],
        out_shape=[sds, sds],
    )(aLu[0], aLu[1], aSu[0], aSu[1], aLi[0], aLi[1], aSi[0], aSi[1],
      agu, agi, upu, upi, u, it)


def _score_body(b1_ref, b2_ref, b3_ref, bl_ref, w_ref, item_ref, out_ref):
    f32 = jnp.float32
    w = w_ref[...]
    ub = (jnp.dot(b1_ref[...][:B], w[:D], preferred_element_type=f32)
          + jnp.dot(b2_ref[...][:B], w[D:2 * D], preferred_element_type=f32)
          + jnp.dot(b3_ref[...][:B], w[2 * D:3 * D],
                    preferred_element_type=f32)
          + jnp.dot(bl_ref[...][:B], w[3 * D:], preferred_element_type=f32))
    out_ref[...] = lax.dot_general(ub, item_ref[...],
                                   (((1,), (1,)), ((), ())),
                                   preferred_element_type=f32)


def _score(b1, b2, b3, bl, wuni, item_emb):
    n_item = item_emb.shape[0]
    IB = 2048
    bsp = pl.BlockSpec((2 * B, D), lambda j: (0, 0))
    return pl.pallas_call(
        _score_body,
        grid=(pl.cdiv(n_item, IB),),
        in_specs=[bsp, bsp, bsp, bsp,
                  pl.BlockSpec(((L + 1) * D, D), lambda j: (0, 0)),
                  pl.BlockSpec((IB, D), lambda j: (j, 0))],
        out_specs=pl.BlockSpec((B, IB), lambda j: (0, j)),
        out_shape=jax.ShapeDtypeStruct((B, n_item), jnp.float32),
    )(b1, b2, b3, bl, wuni, item_emb)


def _pad_idx(idx, n):
    pad = jnp.zeros((n - idx.shape[0],), jnp.int32)
    return jnp.concatenate([idx.astype(jnp.int32), pad])


def kernel(user_ids, item_ids, edge_src_item, edge_dst_user, edge_order_u,
           edge_order_i, last_item_per_user, last_user_per_item, batch_users,
           user_emb, item_emb, user_weight, item_weight, agg_gate_u,
           agg_gate_i, u_te, u_te_k, i_te, i_te_k, last_w_u, last_w_i,
           upd_u, upd_i, unified_map):
    src = edge_src_item.astype(jnp.int32)
    dst = edge_dst_user.astype(jnp.int32)
    t_u = edge_order_u.astype(jnp.int32)
    t_i = edge_order_i.astype(jnp.int32)
    last_it = _pad_idx(last_item_per_user, NP)
    last_us = _pad_idx(last_user_per_item, NP)

    u = _rows(_f32(user_emb), _pad_idx(user_ids, NP))
    i0 = _rows(_f32(item_emb), _pad_idx(item_ids, NP))
    it = i0

    layer_u = []
    for l in range(L):
        g1 = _rows(it, last_it)
        g2 = _rows(u, last_us)
        uh, ih, qlu, qli = _lin(u, it, g1, g2, user_weight[l],
                                item_weight[l], last_w_u[l], last_w_i[l])
        aLu = _agg2(ih, uh, u_te_k[l], u_te[l], src, dst, t_u)
        aSu = _agg2s(ih, qlu, src, dst)
        aLi = _agg2(uh, ih, i_te_k[l], i_te[l], dst, src, t_i)
        aSi = _agg2s(uh, qli, dst, src)
        u, it = _upd(aLu, aSu, aLi, aSi,
                     agg_gate_u[l], agg_gate_i[l], upd_u[l], upd_i[l],
                     u, it)
        layer_u.append(u)

    bidx = jnp.concatenate([batch_users, batch_users]).astype(jnp.int32)
    lidx = jnp.concatenate([last_item_per_user[batch_users]] * 2)
    b1 = _rows(layer_u[0], bidx)
    b2 = _rows(layer_u[1], bidx)
    b3 = _rows(layer_u[2], bidx)
    bl = _rows(i0, lidx.astype(jnp.int32))
    return _score(b1, b2, b3, bl, unified_map, _f32(item_emb))


# speedup vs baseline: 4.5186x; 1.0335x over previous
import functools

import jax
import jax.numpy as jnp
from jax import lax
from jax.experimental import pallas as pl
from jax.experimental.pallas import tpu as pltpu
from jax.experimental.pallas import tpu_sc as plsc

L = 3
D = 128
N = 5000
NP = 5120
E = 320000
TMAX = 50
B = 128

NC = 2
NS = 16
NW = NC * NS
CH = 80
EPW = E // NW
NCHUNK = EPW // CH
DNR = NP // 16
RB = 1024
INV_SCALE = 1.0 / (D ** 0.5)

_mesh = plsc.VectorSubcoreMesh(core_axis_name="c", subcore_axis_name="s")

_GDN = lax.GatherDimensionNumbers(offset_dims=(), collapsed_slice_dims=(0,),
                                  start_index_map=(0,))


def _lanesum(v):
    ii = lax.iota(jnp.int32, 16)
    for k in (1, 2, 4, 8):
        p = lax.gather(v, jnp.bitwise_xor(ii, k)[:, None], _GDN,
                       slice_sizes=(1,),
                       mode=lax.GatherScatterMode.PROMISE_IN_BOUNDS)
        v = v + p
    return v


_lanesum_i = _lanesum


def _f32(x):
    return x.astype(jnp.float32)


@functools.lru_cache(maxsize=None)
def _rows_fn(V, BN):
    bpw = BN // NW
    cw = min(bpw, CH)
    nk = bpw // cw

    @functools.partial(
        pl.kernel,
        out_type=jax.ShapeDtypeStruct((BN, D), jnp.float32),
        mesh=_mesh,
        scratch_types=[
            pltpu.VMEM((cw,), jnp.int32),
            pltpu.VMEM((cw, D), jnp.float32),
            pltpu.SemaphoreType.DMA,
        ],
    )
    def rows_kernel(tab_h, idx_h, out_h, idxv, rbuf, sem):
        wid = lax.axis_index("s") * NC + lax.axis_index("c")
        base = wid * bpw
        for k in range(nk):
            off = base + k * cw
            pltpu.sync_copy(idx_h.at[pl.ds(off, cw)], idxv)
            pltpu.async_copy(tab_h.at[idxv], rbuf, sem).wait()
            pltpu.sync_copy(rbuf, out_h.at[pl.ds(off, cw)])

    return rows_kernel


def _rows(table, idx):
    return _rows_fn(table.shape[0], idx.shape[0])(table, idx)


DR = 48


@functools.lru_cache(maxsize=None)
def _agg_fn(use_te=True):
    nidx = 3 if use_te else 2
    DMA = pltpu.SemaphoreType.DMA
    scratch = (
        [pltpu.VMEM((CH,), jnp.int32)] * nidx
        + [pltpu.VMEM((CH,), jnp.int32)] * nidx
        + [pltpu.VMEM((CH,), jnp.int32)]
        + [pltpu.VMEM((CH, D), jnp.float32)] * (4 if use_te else 2)
        + [pltpu.VMEM((CH, D), jnp.float32)] * 2
        + [pltpu.VMEM_SHARED((NP, D), jnp.float32),
           pltpu.VMEM_SHARED((DR, D), jnp.float32)]
        + [DMA] * (4 if use_te else 2)
        + [DMA] * nidx + [DMA] * nidx
    )
    out_type = (jax.ShapeDtypeStruct((NC, NP, D), jnp.float32),
                jax.ShapeDtypeStruct((NC, DR, D), jnp.float32))

    def body(*refs):
        if use_te:
            (a_h, q_h, tk_h, tv_h, ia_h, iq_h, tt_h, num_h, den_h,
             iaA, iqA, ttA, iaB, iqB, ttB, iqv2, arows, qrows, tkrows,
             tvrows, vbuf, dbuf, num_sh, den_sh,
             semA, semQ, semK, semV, sA1, sA2, sA3, sB1, sB2, sB3) = refs
            setA = (iaA, iqA, ttA, sA1, sA2, sA3)
            setB = (iaB, iqB, ttB, sB1, sB2, sB3)
            idx_hs = (ia_h, iq_h, tt_h)
        else:
            (a_h, q_h, ia_h, iq_h, num_h, den_h,
             iaA, iqA, iaB, iqB, iqv2, arows, qrows, vbuf, dbuf,
             num_sh, den_sh,
             semA, semQ, sA1, sA2, sB1, sB2) = refs
            setA = (iaA, iqA, sA1, sA2)
            setB = (iaB, iqB, sB1, sB2)
            idx_hs = (ia_h, iq_h)
        cid = lax.axis_index("c")
        sid = lax.axis_index("s")
        wid = sid * NC + cid
        ebase = wid * EPW
        iota = lax.iota(jnp.int32, 16)
        zero16 = jnp.zeros((16,), jnp.float32)

        def zrow(r, _):
            for s in range(D // 16):
                vbuf[r, pl.ds(s * 16, 16)] = zero16
            return 0
        lax.fori_loop(0, CH, zrow, 0)

        zb = sid * (NP // NS)
        for k in range(NP // NS // CH):
            pltpu.sync_copy(vbuf, num_sh.at[pl.ds(zb + k * CH, CH)])
        @pl.when(sid < DR // 8)
        def _():
            pltpu.sync_copy(vbuf.at[pl.ds(0, 8)],
                            den_sh.at[pl.ds(sid * 8, 8)])
        plsc.subcore_barrier()

        def fire_idx(c, iset):
            eb = ebase + c * CH
            n = len(idx_hs)
            for h_ref, buf, sem in zip(idx_hs, iset[:n], iset[n:]):
                pltpu.async_copy(h_ref.at[pl.ds(eb, CH)], buf, sem)

        def do_chunk(c, iset, next_c, next_set):
            eb = ebase + c * CH
            n = len(idx_hs)
            for h_ref, buf, sem in zip(idx_hs, iset[:n], iset[n:]):
                pltpu.make_async_copy(h_ref.at[pl.ds(eb, CH)], buf,
                                      sem).wait()
            if next_set is not None:
                fire_idx(next_c, next_set)
            iav = iset[0]
            iqv = iset[1]
            if use_te:
                ttv = iset[2]
            for i in range(CH // 16):
                iq16i = iqv[pl.ds(i * 16, 16)]
                iqv2[pl.ds(i * 16, 16)] = lax.shift_right_logical(iq16i, 7)
            ga = pltpu.async_copy(a_h.at[iav], arows, semA)
            gq = pltpu.async_copy(q_h.at[iqv], qrows, semQ)
            if use_te:
                gk = pltpu.async_copy(tk_h.at[ttv], tkrows, semK)
                gv = pltpu.async_copy(tv_h.at[ttv], tvrows, semV)
            ga.wait()
            gq.wait()
            if use_te:
                gk.wait()
                gv.wait()

            for g in range(CH // 16):
                iq16 = iqv[pl.ds(g * 16, 16)]

                def edge(j, _):
                    jj = g * 16 + j
                    acc1 = zero16
                    for s in range(8):
                        sl = pl.ds(s * 16, 16)
                        kv = arows[jj, sl]
                        if use_te:
                            kv = kv + tkrows[jj, sl]
                        acc1 = acc1 + kv * qrows[jj, sl]
                    w1 = jnp.exp(_lanesum(acc1) * INV_SCALE)
                    for s in range(8):
                        sl = pl.ds(s * 16, 16)
                        vv = arows[jj, sl]
                        if use_te:
                            vv = vv + tvrows[jj, sl]
                        vbuf[jj, sl] = vv * w1
                    sel = jnp.where(iota == j, iq16, 0)
                    s16 = _lanesum_i(sel)
                    c2 = jnp.bitwise_and(s16, D - 1)
                    for s in range(8):
                        m = iota + s * 16 == c2
                        dbuf[jj, pl.ds(s * 16, 16)] = jnp.where(m, w1, 0.0)
                    return 0
                lax.fori_loop(0, 16, edge, 0)

            pltpu.sync_copy(vbuf, num_sh.at[iqv], add=True)
            pltpu.sync_copy(dbuf, den_sh.at[iqv2], add=True)

        fire_idx(0, setA)

        def pair(p, _):
            c0 = 2 * p
            do_chunk(c0, setA, c0 + 1, setB)
            do_chunk(c0 + 1, setB, c0 + 2, setA)
            return 0
        lax.fori_loop(0, NCHUNK // 2, pair, 0)
        do_chunk(NCHUNK - 1, setA, 0, None)

        plsc.subcore_barrier()

        @pl.when(sid == 0)
        def _():
            pltpu.sync_copy(num_sh, num_h.at[cid])
            pltpu.sync_copy(den_sh, den_h.at[cid])

    return pl.kernel(body, out_type=out_type, mesh=_mesh,
                     scratch_types=scratch)


def _agg2(A, Q, TEk, TEv, ia, iq, tt):
    return _agg_fn(True)(A, Q, TEk, TEv, ia, iq, tt)


def _agg2s(A, Q, ia, iq):
    return _agg_fn(False)(A, Q, ia, iq)


def _lin_body(u_ref, it_ref, g1_ref, g2_ref, wu_ref, wi_ref, lwu_ref,
              lwi_ref, uh_ref, ih_ref, qlu_ref, qli_ref):
    f32 = jnp.float32
    uh_ref[...] = jnp.dot(u_ref[...], wu_ref[...], preferred_element_type=f32)
    ih_ref[...] = jnp.dot(it_ref[...], wi_ref[...],
                          preferred_element_type=f32)
    mu = jnp.dot(wi_ref[...], lwu_ref[...], preferred_element_type=f32)
    mi = jnp.dot(wu_ref[...], lwi_ref[...], preferred_element_type=f32)
    qlu_ref[...] = jnp.dot(g1_ref[...], mu, preferred_element_type=f32)
    qli_ref[...] = jnp.dot(g2_ref[...], mi, preferred_element_type=f32)


def _lin(u, it, g1, g2, wu, wi, lwu, lwi):
    row = pl.BlockSpec((RB, D), lambda r: (r, 0))
    wsp = pl.BlockSpec((D, D), lambda r: (0, 0))
    sds = jax.ShapeDtypeStruct((NP, D), jnp.float32)
    return pl.pallas_call(
        _lin_body,
        grid=(NP // RB,),
        in_specs=[row, row, row, row, wsp, wsp, wsp, wsp],
        out_specs=[row, row, row, row],
        out_shape=[sds, sds, sds, sds],
    )(u, it, g1, g2, wu, wi, lwu, lwi)


def _upd_body(nLu, dLu, nSu, dSu, nLi, dLi, nSi, dSi, agu_ref, agi_ref,
              upu_ref, upi_ref, u_ref, it_ref, un_ref, itn_ref):
    f32 = jnp.float32
    ii8 = lax.broadcasted_iota(jnp.int32, (RB, RB // D), 0)
    aa8 = lax.broadcasted_iota(jnp.int32, (RB, RB // D), 1)
    ohr = (ii8 // D == aa8).astype(f32)
    iiD = lax.broadcasted_iota(jnp.int32, (RB, D), 0)
    bbD = lax.broadcasted_iota(jnp.int32, (RB, D), 1)
    ohc = (iiD % D == bbD).astype(f32)

    def h(nref, dref):
        n = nref[...]
        d = dref[...]
        rep = jnp.dot(ohr, d[0] + d[1], preferred_element_type=f32)
        dd = jnp.sum(rep * ohc, axis=1, keepdims=True)
        return (n[0] + n[1]) / (dd + 1e-9)

    hLu = h(nLu, dLu)
    hSu = h(nSu, dSu)
    hLi = h(nLi, dLi)
    hSi = h(nSi, dSi)
    agu = agu_ref[...]
    agi = agi_ref[...]
    upu = upu_ref[...]
    upi = upi_ref[...]
    hu = (jnp.dot(hLu, agu[:D], preferred_element_type=f32)
          + jnp.dot(hSu, agu[D:], preferred_element_type=f32))
    hi = (jnp.dot(hLi, agi[:D], preferred_element_type=f32)
          + jnp.dot(hSi, agi[D:], preferred_element_type=f32))
    un_ref[...] = jnp.tanh(
        jnp.dot(hu, upu[:D], preferred_element_type=f32)
        + jnp.dot(u_ref[...], upu[D:], preferred_element_type=f32))
    itn_ref[...] = jnp.tanh(
        jnp.dot(hi, upi[:D], preferred_element_type=f32)
        + jnp.dot(it_ref[...], upi[D:], preferred_element_type=f32))


def _upd(aLu, aSu, aLi, aSi, agu, agi, upu, upi, u, it):
    nsp = pl.BlockSpec((NC, RB, D), lambda r: (0, r, 0))
    dsp = pl.BlockSpec((NC, RB // D, D), lambda r: (0, r, 0))
    wsp = pl.BlockSpec((2 * D, D), lambda r: (0, 0))
    row = pl.BlockSpec((RB, D), lambda r: (r, 0))
    sds = jax.ShapeDtypeStruct((NP, D), jnp.float32)
    return pl.pallas_call(
        _upd_body,
        grid=(NP // RB,),
        in_specs=[nsp, dsp, nsp, dsp, nsp, dsp, nsp, dsp,
                  wsp, wsp, wsp, wsp, row, row],
        out_specs=[row, row],
        out_shape=[sds, sds],
    )(aLu[0], aLu[1], aSu[0], aSu[1], aLi[0], aLi[1], aSi[0], aSi[1],
      agu, agi, upu, upi, u, it)


def _score_body(b1_ref, b2_ref, b3_ref, bl_ref, w_ref, item_ref, out_ref):
    f32 = jnp.float32
    w = w_ref[...]
    ub = (jnp.dot(b1_ref[...][:B], w[:D], preferred_element_type=f32)
          + jnp.dot(b2_ref[...][:B], w[D:2 * D], preferred_element_type=f32)
          + jnp.dot(b3_ref[...][:B], w[2 * D:3 * D],
                    preferred_element_type=f32)
          + jnp.dot(bl_ref[...][:B], w[3 * D:], preferred_element_type=f32))
    out_ref[...] = lax.dot_general(ub, item_ref[...],
                                   (((1,), (1,)), ((), ())),
                                   preferred_element_type=f32)


def _score(b1, b2, b3, bl, wuni, item_emb):
    n_item = item_emb.shape[0]
    IB = 2048
    bsp = pl.BlockSpec((2 * B, D), lambda j: (0, 0))
    return pl.pallas_call(
        _score_body,
        grid=(pl.cdiv(n_item, IB),),
        in_specs=[bsp, bsp, bsp, bsp,
                  pl.BlockSpec(((L + 1) * D, D), lambda j: (0, 0)),
                  pl.BlockSpec((IB, D), lambda j: (j, 0))],
        out_specs=pl.BlockSpec((B, IB), lambda j: (0, j)),
        out_shape=jax.ShapeDtypeStruct((B, n_item), jnp.float32),
    )(b1, b2, b3, bl, wuni, item_emb)


def _pad_idx(idx, n):
    pad = jnp.zeros((n - idx.shape[0],), jnp.int32)
    return jnp.concatenate([idx.astype(jnp.int32), pad])


def kernel(user_ids, item_ids, edge_src_item, edge_dst_user, edge_order_u,
           edge_order_i, last_item_per_user, last_user_per_item, batch_users,
           user_emb, item_emb, user_weight, item_weight, agg_gate_u,
           agg_gate_i, u_te, u_te_k, i_te, i_te_k, last_w_u, last_w_i,
           upd_u, upd_i, unified_map):
    src = edge_src_item.astype(jnp.int32)
    dst = edge_dst_user.astype(jnp.int32)
    t_u = edge_order_u.astype(jnp.int32)
    t_i = edge_order_i.astype(jnp.int32)
    last_it = _pad_idx(last_item_per_user, NP)
    last_us = _pad_idx(last_user_per_item, NP)

    u = _rows(_f32(user_emb), _pad_idx(user_ids, NP))
    i0 = _rows(_f32(item_emb), _pad_idx(item_ids, NP))
    it = i0

    layer_u = []
    for l in range(L):
        g1 = _rows(it, last_it)
        g2 = _rows(u, last_us)
        uh, ih, qlu, qli = _lin(u, it, g1, g2, user_weight[l],
                                item_weight[l], last_w_u[l], last_w_i[l])
        aLu = _agg2(ih, uh, u_te_k[l], u_te[l], src, dst, t_u)
        aSu = _agg2s(ih, qlu, src, dst)
        aLi = _agg2(uh, ih, i_te_k[l], i_te[l], dst, src, t_i)
        aSi = _agg2s(uh, qli, dst, src)
        u, it = _upd(aLu, aSu, aLi, aSi,
                     agg_gate_u[l], agg_gate_i[l], upd_u[l], upd_i[l],
                     u, it)
        layer_u.append(u)

    bidx = jnp.concatenate([batch_users, batch_users]).astype(jnp.int32)
    lidx = jnp.concatenate([last_item_per_user[batch_users]] * 2)
    b1 = _rows(layer_u[0], bidx)
    b2 = _rows(layer_u[1], bidx)
    b3 = _rows(layer_u[2], bidx)
    bl = _rows(i0, lidx.astype(jnp.int32))
    return _score(b1, b2, b3, bl, unified_map, _f32(item_emb))
